# baseline (speedup 1.0000x reference)
import jax
import jax.numpy as jnp
from jax.experimental import pallas as pl
from jax.experimental.pallas import tpu as pltpu

_B, _C_IN, _H, _W = 32, 96, 56, 56
_C_OUT = 128


def _compact_scale_row(w_row, w_col):
    lane = jax.lax.broadcasted_iota(jnp.int32, (_C_OUT, _C_OUT), 1)
    sub = jax.lax.broadcasted_iota(jnp.int32, (_C_OUT, _C_OUT), 0)
    incl = jnp.where((lane <= sub) & (w_row != 0.0), 1.0, 0.0)
    cs_col = jnp.sum(incl, axis=1, keepdims=True)
    lanef = (lane + 1).astype(jnp.float32)
    pick = (cs_col == lanef) & (w_col != 0.0)
    return jnp.sum(jnp.where(pick, w_col, 0.0), axis=0, keepdims=True)


def _body(w_row_ref, w_col_ref, in_ref, out_ref):
    s_row = _compact_scale_row(w_row_ref[:, :], w_col_ref[:, :])
    scale = s_row[:, 0:_C_IN].reshape(1, 1, _C_IN)
    out_ref[0, :, :, 0:_C_IN] = in_ref[0, :, :, :] * scale
    out_ref[0, :, :, _C_IN:_C_OUT] = jnp.zeros(
        (_H // 2, _W, _C_OUT - _C_IN), dtype=out_ref.dtype
    )


def kernel(input, weight_kse, weight):
    del weight_kse
    w_row = weight.reshape(1, _C_OUT)
    w_col = weight.reshape(_C_OUT, 1)
    xt = jnp.transpose(input, (0, 2, 3, 1))

    out_t = pl.pallas_call(
        _body,
        grid=(_B, 2),
        in_specs=[
            pl.BlockSpec((1, _C_OUT), lambda b, h: (0, 0)),
            pl.BlockSpec((_C_OUT, 1), lambda b, h: (0, 0)),
            pl.BlockSpec((1, _H // 2, _W, _C_IN), lambda b, h: (b, h, 0, 0)),
        ],
        out_specs=pl.BlockSpec(
            (1, _H // 2, _W, _C_OUT), lambda b, h: (b, h, 0, 0)
        ),
        out_shape=jax.ShapeDtypeStruct((_B, _H, _W, _C_OUT), input.dtype),
        compiler_params=pltpu.CompilerParams(
            dimension_semantics=("arbitrary", "arbitrary"),
        ),
    )(w_row, w_col, xt)
    return jnp.transpose(out_t, (0, 3, 1, 2))


# speedup vs baseline: 1.5056x; 1.5056x over previous
import jax
import jax.numpy as jnp
from jax.experimental import pallas as pl
from jax.experimental.pallas import tpu as pltpu

_B, _C_IN, _H, _W = 32, 96, 56, 56
_C_OUT = 128


def _compact_scale_row(w_row, w_col):
    lane = jax.lax.broadcasted_iota(jnp.int32, (_C_OUT, _C_OUT), 1)
    sub = jax.lax.broadcasted_iota(jnp.int32, (_C_OUT, _C_OUT), 0)
    incl = jnp.where((lane <= sub) & (w_row != 0.0), 1.0, 0.0)
    cs_col = jnp.sum(incl, axis=1, keepdims=True)
    lanef = (lane + 1).astype(jnp.float32)
    pick = (cs_col == lanef) & (w_col != 0.0)
    return jnp.sum(jnp.where(pick, w_col, 0.0), axis=0, keepdims=True)


def _body(w_row_ref, w_col_ref, in_ref, out_ref, s_ref):
    @pl.when(pl.program_id(0) == 0)
    def _():
        s_ref[:, :] = _compact_scale_row(w_row_ref[:, :], w_col_ref[:, :])

    scale = s_ref[:, 0:_C_IN].reshape(1, 1, _C_IN)
    out_ref[0, :, :, 0:_C_IN] = in_ref[0, :, :, :] * scale
    out_ref[0, :, :, _C_IN:_C_OUT] = jnp.zeros(
        (_H, _W, _C_OUT - _C_IN), dtype=out_ref.dtype
    )


def kernel(input, weight_kse, weight):
    del weight_kse
    w_row = weight.reshape(1, _C_OUT)
    w_col = weight.reshape(_C_OUT, 1)
    xt = jnp.transpose(input, (0, 2, 3, 1))

    out_t = pl.pallas_call(
        _body,
        grid=(_B,),
        in_specs=[
            pl.BlockSpec((1, _C_OUT), lambda b: (0, 0)),
            pl.BlockSpec((_C_OUT, 1), lambda b: (0, 0)),
            pl.BlockSpec((1, _H, _W, _C_IN), lambda b: (b, 0, 0, 0)),
        ],
        out_specs=pl.BlockSpec((1, _H, _W, _C_OUT), lambda b: (b, 0, 0, 0)),
        out_shape=jax.ShapeDtypeStruct((_B, _H, _W, _C_OUT), input.dtype),
        scratch_shapes=[pltpu.VMEM((1, _C_OUT), jnp.float32)],
        compiler_params=pltpu.CompilerParams(
            dimension_semantics=("arbitrary",),
        ),
    )(w_row, w_col, xt)
    return jnp.transpose(out_t, (0, 3, 1, 2))


# speedup vs baseline: 1.7617x; 1.1701x over previous
import jax
import jax.numpy as jnp
from jax.experimental import pallas as pl
from jax.experimental.pallas import tpu as pltpu

_B, _C_IN, _H, _W = 32, 96, 56, 56
_C_OUT = 128


def _compact_scale_row(w_row, w_col):
    lane = jax.lax.broadcasted_iota(jnp.int32, (_C_OUT, _C_OUT), 1)
    sub = jax.lax.broadcasted_iota(jnp.int32, (_C_OUT, _C_OUT), 0)
    incl = jnp.where((lane <= sub) & (w_row != 0.0), 1.0, 0.0)
    cs_col = jnp.sum(incl, axis=1, keepdims=True)
    lanef = (lane + 1).astype(jnp.float32)
    pick = (cs_col == lanef) & (w_col != 0.0)
    return jnp.sum(jnp.where(pick, w_col, 0.0), axis=0, keepdims=True)


def _body(w_row_ref, w_col_ref, in_ref, out_ref, s_ref):
    @pl.when(pl.program_id(0) == 0)
    def _():
        s_ref[:, :] = _compact_scale_row(w_row_ref[:, :], w_col_ref[:, :])

    scale = s_ref[:, 0:_C_IN].reshape(1, 1, _C_IN)
    out_ref[:, :, :, 0:_C_IN] = in_ref[:, :, :, :] * scale.reshape(1, 1, 1, _C_IN)
    out_ref[:, :, :, _C_IN:_C_OUT] = jnp.zeros(
        (2, _H, _W, _C_OUT - _C_IN), dtype=out_ref.dtype
    )


def kernel(input, weight_kse, weight):
    del weight_kse
    w_row = weight.reshape(1, _C_OUT)
    w_col = weight.reshape(_C_OUT, 1)
    xt = jnp.transpose(input, (0, 2, 3, 1))

    out_t = pl.pallas_call(
        _body,
        grid=(_B // 2,),
        in_specs=[
            pl.BlockSpec((1, _C_OUT), lambda b: (0, 0)),
            pl.BlockSpec((_C_OUT, 1), lambda b: (0, 0)),
            pl.BlockSpec((2, _H, _W, _C_IN), lambda b: (b, 0, 0, 0)),
        ],
        out_specs=pl.BlockSpec((2, _H, _W, _C_OUT), lambda b: (b, 0, 0, 0)),
        out_shape=jax.ShapeDtypeStruct((_B, _H, _W, _C_OUT), input.dtype),
        scratch_shapes=[pltpu.VMEM((1, _C_OUT), jnp.float32)],
        compiler_params=pltpu.CompilerParams(
            dimension_semantics=("arbitrary",),
        ),
    )(w_row, w_col, xt)
    return jnp.transpose(out_t, (0, 3, 1, 2))


# speedup vs baseline: 1.8440x; 1.0467x over previous
import jax
import jax.numpy as jnp
from jax.experimental import pallas as pl
from jax.experimental.pallas import tpu as pltpu

_B, _C_IN, _H, _W = 32, 96, 56, 56
_C_OUT = 128


def _compact_scale_row(w_row, w_col):
    lane = jax.lax.broadcasted_iota(jnp.int32, (_C_OUT, _C_OUT), 1)
    sub = jax.lax.broadcasted_iota(jnp.int32, (_C_OUT, _C_OUT), 0)
    incl = jnp.where((lane <= sub) & (w_row != 0.0), 1.0, 0.0)
    cs_col = jnp.sum(incl, axis=1, keepdims=True)
    lanef = (lane + 1).astype(jnp.float32)
    pick = (cs_col == lanef) & (w_col != 0.0)
    return jnp.sum(jnp.where(pick, w_col, 0.0), axis=0, keepdims=True)


def _body(w_row_ref, w_col_ref, in_ref, out_ref, s_ref):
    @pl.when(pl.program_id(0) == 0)
    def _():
        s_ref[:, :] = _compact_scale_row(w_row_ref[:, :], w_col_ref[:, :])

    scale = s_ref[:, 0:_C_IN].reshape(1, 1, _C_IN)
    out_ref[:, :, :, 0:_C_IN] = in_ref[:, :, :, :] * scale.reshape(1, 1, 1, _C_IN)
    out_ref[:, :, :, _C_IN:_C_OUT] = jnp.zeros(
        (4, _H, _W, _C_OUT - _C_IN), dtype=out_ref.dtype
    )


def kernel(input, weight_kse, weight):
    del weight_kse
    w_row = weight.reshape(1, _C_OUT)
    w_col = weight.reshape(_C_OUT, 1)
    xt = jnp.transpose(input, (0, 2, 3, 1))

    out_t = pl.pallas_call(
        _body,
        grid=(_B // 4,),
        in_specs=[
            pl.BlockSpec((1, _C_OUT), lambda b: (0, 0)),
            pl.BlockSpec((_C_OUT, 1), lambda b: (0, 0)),
            pl.BlockSpec((4, _H, _W, _C_IN), lambda b: (b, 0, 0, 0)),
        ],
        out_specs=pl.BlockSpec((4, _H, _W, _C_OUT), lambda b: (b, 0, 0, 0)),
        out_shape=jax.ShapeDtypeStruct((_B, _H, _W, _C_OUT), input.dtype),
        scratch_shapes=[pltpu.VMEM((1, _C_OUT), jnp.float32)],
        compiler_params=pltpu.CompilerParams(
            dimension_semantics=("arbitrary",),
        ),
    )(w_row, w_col, xt)
    return jnp.transpose(out_t, (0, 3, 1, 2))


# speedup vs baseline: 1.9557x; 1.0606x over previous
import jax
import jax.numpy as jnp
from jax.experimental import pallas as pl
from jax.experimental.pallas import tpu as pltpu

_B, _C_IN, _H, _W = 32, 96, 56, 56
_C_OUT = 128


def _compact_scale_row(w_row, w_col):
    lane = jax.lax.broadcasted_iota(jnp.int32, (_C_OUT, _C_OUT), 1)
    sub = jax.lax.broadcasted_iota(jnp.int32, (_C_OUT, _C_OUT), 0)
    incl = jnp.where((lane <= sub) & (w_row != 0.0), 1.0, 0.0)
    cs_col = jnp.sum(incl, axis=1, keepdims=True)
    lanef = (lane + 1).astype(jnp.float32)
    pick = (cs_col == lanef) & (w_col != 0.0)
    return jnp.sum(jnp.where(pick, w_col, 0.0), axis=0, keepdims=True)


def _body(w_row_ref, w_col_ref, in_ref, out_ref, s_ref):
    @pl.when(pl.program_id(0) == 0)
    def _():
        s_ref[:, :] = _compact_scale_row(w_row_ref[:, :], w_col_ref[:, :])

    scale = s_ref[:, 0:_C_IN].reshape(1, 1, _C_IN)
    out_ref[:, :, :, 0:_C_IN] = in_ref[:, :, :, :] * scale.reshape(1, 1, 1, _C_IN)
    out_ref[:, :, :, _C_IN:_C_OUT] = jnp.zeros(
        (8, _H, _W, _C_OUT - _C_IN), dtype=out_ref.dtype
    )


def kernel(input, weight_kse, weight):
    del weight_kse
    w_row = weight.reshape(1, _C_OUT)
    w_col = weight.reshape(_C_OUT, 1)
    xt = jnp.transpose(input, (0, 2, 3, 1))

    out_t = pl.pallas_call(
        _body,
        grid=(_B // 8,),
        in_specs=[
            pl.BlockSpec((1, _C_OUT), lambda b: (0, 0)),
            pl.BlockSpec((_C_OUT, 1), lambda b: (0, 0)),
            pl.BlockSpec((8, _H, _W, _C_IN), lambda b: (b, 0, 0, 0)),
        ],
        out_specs=pl.BlockSpec((8, _H, _W, _C_OUT), lambda b: (b, 0, 0, 0)),
        out_shape=jax.ShapeDtypeStruct((_B, _H, _W, _C_OUT), input.dtype),
        scratch_shapes=[pltpu.VMEM((1, _C_OUT), jnp.float32)],
        compiler_params=pltpu.CompilerParams(
            dimension_semantics=("arbitrary",),
        ),
    )(w_row, w_col, xt)
    return jnp.transpose(out_t, (0, 3, 1, 2))
